# Initial kernel scaffold; baseline (speedup 1.0000x reference)
import jax
import jax.numpy as jnp
from jax.experimental import pallas as pl


def kernel(h, edge_index, e, params):
    raise NotImplementedError("write your pallas kernel here")


# speedup vs baseline: 2.6188x; 2.6188x over previous
import functools

import jax
import jax.numpy as jnp
from jax import lax
from jax.experimental import pallas as pl
from jax.experimental.pallas import tpu as pltpu
from jax.experimental.pallas import tpu_sc as plsc

N = 10000
E = 320000
H = 128
L = 4
P = L + 1

ROW_BLK = 1000

NC = 2
NS = 16
NW = NC * NS
EW = E // NW
K = 80
NB = EW // K
NP = 10240
NROWS_T = NP // NS


def _emb_body(h_ref, w_ref, b_ref, o_ref):
    o_ref[...] = (
        jnp.dot(h_ref[...], w_ref[...], preferred_element_type=jnp.float32)
        + b_ref[...]
    )


def _tc_embed(h, w, b):
    return pl.pallas_call(
        _emb_body,
        grid=(N // ROW_BLK,),
        in_specs=[
            pl.BlockSpec((ROW_BLK, H), lambda i: (i, 0)),
            pl.BlockSpec((H, H), lambda i: (0, 0)),
            pl.BlockSpec((1, H), lambda i: (0, 0)),
        ],
        out_specs=pl.BlockSpec((ROW_BLK, H), lambda i: (i, 0)),
        out_shape=jax.ShapeDtypeStruct((N, H), jnp.float32),
    )(h, w, b.reshape(1, H))


def _layer_body(x_ref, n0_ref, n1_ref, w0_ref, c0_ref, w1_ref, b1_ref, ee_ref,
                ga_ref, ca_ref, gn_ref, cn_ref, o_ref):
    x = x_ref[...]
    t = ee_ref[...] * x + n0_ref[...] + n1_ref[...]
    z1 = jnp.maximum(
        jnp.dot(t, w0_ref[...], preferred_element_type=jnp.float32) + c0_ref[...],
        0.0,
    )
    z2 = jnp.dot(z1, w1_ref[...], preferred_element_type=jnp.float32) + b1_ref[...]
    y = jnp.maximum(ga_ref[...] * z2 + ca_ref[...], 0.0)
    t2 = jnp.maximum(gn_ref[...] * y + cn_ref[...], 0.0)
    o_ref[...] = x + t2


def _tc_layer(x, n0, n1, w0, c0, w1, b1, ee, ga, ca, gn, cn):
    vec = lambda v: v.reshape(1, H)
    return pl.pallas_call(
        _layer_body,
        grid=(N // ROW_BLK,),
        in_specs=[
            pl.BlockSpec((ROW_BLK, H), lambda i: (i, 0)),
            pl.BlockSpec((ROW_BLK, H), lambda i: (i, 0)),
            pl.BlockSpec((ROW_BLK, H), lambda i: (i, 0)),
            pl.BlockSpec((H, H), lambda i: (0, 0)),
            pl.BlockSpec((1, H), lambda i: (0, 0)),
            pl.BlockSpec((H, H), lambda i: (0, 0)),
            pl.BlockSpec((1, H), lambda i: (0, 0)),
            pl.BlockSpec((1, 1), lambda i: (0, 0)),
            pl.BlockSpec((1, H), lambda i: (0, 0)),
            pl.BlockSpec((1, H), lambda i: (0, 0)),
            pl.BlockSpec((1, H), lambda i: (0, 0)),
            pl.BlockSpec((1, H), lambda i: (0, 0)),
        ],
        out_specs=pl.BlockSpec((ROW_BLK, H), lambda i: (i, 0)),
        out_shape=jax.ShapeDtypeStruct((N, H), jnp.float32),
    )(x, n0, n1, w0, vec(c0), w1, vec(b1), ee.reshape(1, 1), vec(ga), vec(ca),
      vec(gn), vec(cn))


def _pack_body(x_ref, w_ref, b_ref, oa_ref, ob_ref):
    ab = (
        jnp.dot(x_ref[...], w_ref[...], preferred_element_type=jnp.float32)
        + b_ref[...]
    )
    oa_ref[...] = ab[:, :H]
    ob_ref[...] = ab[:, H:]


def _tc_pack(x, wab, bab, li):
    return pl.pallas_call(
        _pack_body,
        grid=(N // ROW_BLK,),
        in_specs=[
            pl.BlockSpec((ROW_BLK, H), lambda i: (i, 0)),
            pl.BlockSpec((H, 2 * H), lambda i: (0, 0)),
            pl.BlockSpec((1, 2 * H), lambda i: (0, 0)),
        ],
        out_specs=[
            pl.BlockSpec((ROW_BLK, H), lambda i: (i, 0)),
            pl.BlockSpec((ROW_BLK, H), lambda i: (i, 0)),
        ],
        out_shape=[
            jax.ShapeDtypeStruct((N, H), jnp.float32),
            jax.ShapeDtypeStruct((N, H), jnp.float32),
        ],
    )(x, wab, bab.reshape(1, 2 * H))


def _segsum_body(x_ref, src_ref, dst_ref, out_ref,
                 idxs_v, idxd_v, rows_v, zero_v, shared, sem):
    c = lax.axis_index("c")
    s = lax.axis_index("s")
    wid = s * NC + c
    base = wid * EW

    zsp = jnp.zeros((16,), jnp.float32)

    def zrow(r, carry):
        for cc in range(H // 16):
            zero_v[r, pl.ds(cc * 16, 16)] = zsp
        return carry

    lax.fori_loop(0, NROWS_T // 5, zrow, 0)
    for b in range(5):
        pltpu.sync_copy(zero_v, shared.at[pl.ds(s * NROWS_T + b * (NROWS_T // 5),
                                                NROWS_T // 5)])
    plsc.subcore_barrier()

    def block(j, carry):
        eb = base + j * K
        pltpu.sync_copy(src_ref.at[pl.ds(eb, K)], idxs_v)
        pltpu.sync_copy(dst_ref.at[pl.ds(eb, K)], idxd_v)
        pltpu.async_copy(x_ref.at[idxs_v], rows_v, sem).wait()
        pltpu.sync_copy(rows_v, shared.at[idxd_v], add=True)
        return carry

    lax.fori_loop(0, NB, block, 0)
    plsc.subcore_barrier()

    pltpu.sync_copy(shared.at[pl.ds(s * NROWS_T, NROWS_T)],
                    out_ref.at[c, pl.ds(s * NROWS_T, NROWS_T)])


def _sc_segsum(x, src, dst):
    mesh = plsc.VectorSubcoreMesh(core_axis_name="c", subcore_axis_name="s",
                                  num_cores=NC, num_subcores=NS)
    f = pl.kernel(
        _segsum_body,
        out_type=jax.ShapeDtypeStruct((NC, NP, H), jnp.float32),
        mesh=mesh,
        scratch_types=[
            pltpu.VMEM((K,), jnp.int32),
            pltpu.VMEM((K,), jnp.int32),
            pltpu.VMEM((K, H), jnp.float32),
            pltpu.VMEM((NROWS_T // 5, H), jnp.float32),
            pltpu.VMEM_SHARED((NP, H), jnp.float32),
            pltpu.SemaphoreType.DMA,
        ],
        compiler_params=pltpu.CompilerParams(needs_layout_passes=False),
    )
    return f(x, src, dst)


W_U = 0
W_V = P * H
W_W2 = 2 * P * H
W_ACCI = 3 * P * H
W_LEN = 3 * P * H + 16


def _scorer_body(ap_ref, bp_ref, src_ref, dst_ref, e_ref, w_ref, out_ref,
                 idxs_v, idxd_v, e_v, rowsA, rowsB, acc_v, wts_v, out_v,
                 semA, semB):
    c = lax.axis_index("c")
    s = lax.axis_index("s")
    wid = s * NC + c
    base = wid * EW

    pltpu.sync_copy(w_ref, wts_v)
    acci = wts_v[pl.ds(W_ACCI, 16)]
    col0 = jnp.full((16,), 0, jnp.int32)
    col1 = jnp.full((16,), 1, jnp.int32)
    lane0 = lax.iota(jnp.int32, 16) == col0

    def block(j, carry):
        eb = base + j * K
        pltpu.sync_copy(src_ref.at[pl.ds(eb, K)], idxs_v)
        pltpu.sync_copy(dst_ref.at[pl.ds(eb, K)], idxd_v)
        pltpu.sync_copy(e_ref.at[pl.ds(2 * eb, 2 * K)], e_v)
        cpa = pltpu.async_copy(ap_ref.at[idxs_v], rowsA, semA)
        cpb = pltpu.async_copy(bp_ref.at[idxd_v], rowsB, semB)
        cpa.wait()
        cpb.wait()

        def init_i(i, carry2):
            acc_v[i, :] = acci
            return carry2

        lax.fori_loop(0, K, init_i, 0)

        for l in range(P):
            us = [wts_v[pl.ds(W_U + l * H + cc * 16, 16)] for cc in range(H // 16)]
            vs = [wts_v[pl.ds(W_V + l * H + cc * 16, 16)] for cc in range(H // 16)]
            ws = [wts_v[pl.ds(W_W2 + l * H + cc * 16, 16)] for cc in range(H // 16)]

            def edge_i(i, carry2, l=l, us=us, vs=vs, ws=ws):
                si = jnp.full((16,), 2 * i, jnp.int32)
                e0 = plsc.load_gather(e_v, [si + col0])
                e1 = plsc.load_gather(e_v, [si + col1])
                acc = acc_v[i, :]
                for cc in range(H // 16):
                    a = rowsA[i, pl.ds(l * H + cc * 16, 16)]
                    b = rowsB[i, pl.ds(l * H + cc * 16, 16)]
                    sv = a + b + e0 * us[cc] + e1 * vs[cc]
                    acc = acc + jnp.maximum(sv, 0.0) * ws[cc]
                acc_v[i, :] = acc
                return carry2

            lax.fori_loop(0, K, edge_i, 0)

        def fin_i(i, carry2):
            ssum = jnp.sum(acc_v[i, :])
            val = jnp.full((16,), jnp.maximum(ssum, 0.0), jnp.float32)
            si = jnp.full((16,), i, jnp.int32)
            plsc.store_scatter(out_v, [si], val, mask=lane0)
            return carry2

        lax.fori_loop(0, K, fin_i, 0)
        pltpu.sync_copy(out_v, out_ref.at[pl.ds(eb, K)])
        return carry

    lax.fori_loop(0, NB, block, 0)


def _sc_score(apack, bpack, src, dst, e, wts):
    mesh = plsc.VectorSubcoreMesh(core_axis_name="c", subcore_axis_name="s",
                                  num_cores=NC, num_subcores=NS)
    f = pl.kernel(
        _scorer_body,
        out_type=jax.ShapeDtypeStruct((E,), jnp.float32),
        mesh=mesh,
        scratch_types=[
            pltpu.VMEM((K,), jnp.int32),
            pltpu.VMEM((K,), jnp.int32),
            pltpu.VMEM((2 * K,), jnp.float32),
            pltpu.VMEM((K, P * H), jnp.float32),
            pltpu.VMEM((K, P * H), jnp.float32),
            pltpu.VMEM((K, 16), jnp.float32),
            pltpu.VMEM((W_LEN,), jnp.float32),
            pltpu.VMEM((K,), jnp.float32),
            pltpu.SemaphoreType.DMA,
            pltpu.SemaphoreType.DMA,
        ],
        compiler_params=pltpu.CompilerParams(needs_layout_passes=False),
    )
    return f(apack, bpack, src, dst, e.reshape(E * 2), wts)


def kernel(h, edge_index, e, params):
    src = edge_index[0]
    dst = edge_index[1]
    bn_s = 1.0 / jnp.sqrt(jnp.float32(1.0 + 1e-5))

    x = _tc_embed(h, params['emb_W'], params['emb_b'])

    xs = [x]
    for lp in params['layers']:
        s0 = bn_s * lp['bn0_g']
        w0f = lp['W0'] * s0[None, :]
        c0 = lp['b0'] * s0 + lp['bn0_b']
        ga = bn_s * lp['bn_apply_g']
        ca = lp['bn_apply_b']
        gn = bn_s * lp['bn_node_g']
        cn = lp['bn_node_b']
        nacc = _sc_segsum(x, src, dst)
        x = _tc_layer(x, nacc[0, :N], nacc[1, :N], w0f, c0, lp['W1'], lp['b1'],
                      (1.0 + lp['eps']).reshape(()), ga, ca, gn, cn)
        xs.append(x)

    aparts, bparts = [], []
    for xl, pp in zip(xs, params['preds']):
        wab = jnp.concatenate([pp['W1'][:H], pp['W1'][H:2 * H]], axis=1)
        bab = jnp.concatenate([pp['b1'], jnp.zeros((H,), jnp.float32)])
        al, bl = _tc_pack(xl, wab, bab, 0)
        aparts.append(al)
        bparts.append(bl)
    apack = jnp.concatenate(aparts, axis=1)
    bpack = jnp.concatenate(bparts, axis=1)

    us = jnp.concatenate([pp['W1'][2 * H] for pp in params['preds']])
    vs = jnp.concatenate([pp['W1'][2 * H + 1] for pp in params['preds']])
    w2 = jnp.concatenate([pp['W2'][:, 0] for pp in params['preds']])
    acci = jnp.zeros((16,), jnp.float32).at[0].set(
        sum(pp['b2'][0] for pp in params['preds']))
    wts = jnp.concatenate([us, vs, w2, acci])

    score = _sc_score(apack, bpack, src, dst, e, wts)
    return score[:, None]
